# baseline (speedup 1.0000x reference)
import functools

import jax
import jax.numpy as jnp
from jax import lax
from jax.experimental import pallas as pl
from jax.experimental.pallas import tpu as pltpu
from jax.experimental.pallas import tpu_sc as plsc

N_NODES = 10000
HID = 256
H1 = 4
H2 = 1

L = 16
NTILES = 32
NPT = 313
RPW = 352
PTS = NPT + 16
NEG = -1e30


def _mm_kernel(x_ref, w_ref, b_ref, o_ref):
    o_ref[...] = jnp.dot(x_ref[...], w_ref[...],
                         preferred_element_type=jnp.float32) + b_ref[...]


def _matmul_bias(x, w, b, block_m=400):
    M, K = x.shape
    _, Nf = w.shape
    return pl.pallas_call(
        _mm_kernel,
        grid=(M // block_m,),
        in_specs=[
            pl.BlockSpec((block_m, K), lambda i: (i, 0)),
            pl.BlockSpec((K, Nf), lambda i: (0, 0)),
            pl.BlockSpec((1, Nf), lambda i: (0, 0)),
        ],
        out_specs=pl.BlockSpec((block_m, Nf), lambda i: (i, 0)),
        out_shape=jax.ShapeDtypeStruct((M, Nf), jnp.float32),
    )(x, w, b[None])


def _edge_body(D, H, EC, NB, xl_hbm, xrf_hbm, srcp_hbm, rpt_hbm, att6_hbm,
               att4_hbm, bias_hbm, out_hbm, rp_v, att6_v, att4_v, bias_v,
               xr_b, ids_v, rows_v, lbuf_v, wbuf_v, acc_b, sem_g, sem_x):
    HD = D // H
    JH = HD // L
    JD = D // L
    ECH = EC // L

    c = lax.axis_index("c")
    s = lax.axis_index("s")
    w = s * 2 + c
    obase = w * PTS

    pltpu.sync_copy(rpt_hbm.at[w], rp_v)
    pltpu.sync_copy(att6_hbm, att6_v)
    pltpu.sync_copy(att4_hbm, att4_v)
    pltpu.sync_copy(bias_hbm, bias_v)
    lanes = lax.iota(jnp.int32, L)

    def _sv(ref, i):
        return ref[pl.ds(i, L)][0]

    def _stage_ids(e_lo):
        al = pl.multiple_of((e_lo // 8) * 8, 8)
        pltpu.sync_copy(srcp_hbm.at[pl.ds(al, EC + 8)],
                        ids_v.at[pl.ds(0, EC + 8)])
        return e_lo - al

    def _gather_chunk(off0, c0, valid):
        pos = c0 * L + lanes
        mask = pos < valid
        offs = jnp.where(mask, off0 + pos, 0)
        ids = plsc.load_gather(ids_v, [offs])
        ids = jnp.where(mask, ids, 0)
        return pltpu.async_copy(xl_hbm.at[ids],
                                rows_v.at[pl.ds(c0 * L, L)], sem_g)

    def _logits(local_lo, cnt, xrb):
        for h in range(H):
            for cc in range(ECH):
                lbuf_v[h, cc * L:(cc + 1) * L] = jnp.full(
                    (L,), NEG, jnp.float32)
        n_ch = (cnt + L - 1) // L

        def ch_body(ch, _):
            cbase = ch * L
            maskv = (cbase + lanes) < cnt
            rvec = jnp.where(maskv, local_lo + cbase + lanes, local_lo)
            for h in range(H):
                def j_body(j2, acc):
                    colb = h * HD + j2 * L
                    xrch = xr_b[pl.ds(xrb + colb, L)]
                    a6ch = att6_v[pl.ds(colb, L)]
                    a4ch = att4_v[pl.ds(colb, L)]
                    for kk in range(L):
                        colv = plsc.load_gather(
                            rows_v,
                            [rvec, jnp.full((L,), colb + kk, jnp.int32)])
                        sv = colv + jnp.full((L,), xrch[kk])
                        acc = acc + jnp.full((L,), a6ch[kk]) * sv
                        acc = acc + jnp.full((L,), a4ch[kk]) * jnp.abs(sv)
                    return acc

                lg = lax.fori_loop(0, JH, j_body,
                                   jnp.zeros((L,), jnp.float32))
                lbuf_v[h, pl.ds(cbase, L)] = jnp.where(
                    maskv, lg, jnp.full((L,), NEG, jnp.float32))
            return 0

        lax.fori_loop(0, n_ch, ch_body, 0)

    def _weights(m_old, denv_old):
        m_new, denv_new, scvs = [], [], []
        for h in range(H):
            mh = m_old[h]
            for cc in range(ECH):
                mh = jnp.maximum(mh, jnp.max(lbuf_v[h, cc * L:(cc + 1) * L]))
            scv = jnp.exp(jnp.full((L,), m_old[h] - mh))
            denv = denv_old[h] * scv
            for cc in range(ECH):
                wv = jnp.exp(lbuf_v[h, cc * L:(cc + 1) * L] - mh)
                wbuf_v[h, cc * L:(cc + 1) * L] = wv
                denv = denv + wv
            m_new.append(mh)
            denv_new.append(denv)
            scvs.append(scv)
        return tuple(m_new), tuple(denv_new), tuple(scvs)

    def _accumulate(local_lo, cnt, ab):
        def e2_body(el, _):
            er = local_lo + el
            for h in range(H):
                wv = plsc.load_gather(
                    wbuf_v,
                    [jnp.full((L,), h, jnp.int32),
                     jnp.full((L,), el, jnp.int32)])
                for j in range(JH):
                    col = h * HD + j * L
                    acc_b[pl.ds(ab + col, L)] = (
                        acc_b[pl.ds(ab + col, L)]
                        + wv * rows_v[er, col:col + L])
            return 0

        lax.fori_loop(0, cnt, e2_body, 0)

    def _finalize(denv_fin, ab):
        for h in range(H):
            inv = jnp.full((L,), 1.0) / jnp.full((L,), jnp.sum(denv_fin[h]))
            for j in range(JH):
                col = h * HD + j * L
                ov = acc_b[pl.ds(ab + col, L)] * inv + bias_v[col:col + L]
                acc_b[pl.ds(ab + col, L)] = jnp.maximum(ov, 0.0)

    def _zero_acc(ab):
        for j in range(JD):
            acc_b[pl.ds(ab + j * L, L)] = jnp.zeros((L,), jnp.float32)

    def _big_node(ni, e_lo, cnt):
        node = w * NPT + ni
        pltpu.async_copy(xrf_hbm.at[pl.ds(node * D, D)],
                         xr_b.at[pl.ds(0, D)], sem_x).wait()
        _zero_acc(0)
        n_mc = (cnt + EC - 1) // EC

        def mc_body(mc, mc_carry):
            m_old, denv_old = mc_carry
            e_base = e_lo + mc * EC
            rem_mc = jnp.minimum(cnt - mc * EC, EC)
            off0 = _stage_ids(e_base)
            cps = [_gather_chunk(off0, cc, rem_mc) for cc in range(ECH)]
            for cp in cps:
                cp.wait()
            _logits(0, rem_mc, 0)
            m_new, denv_new, scvs = _weights(m_old, denv_old)

            @pl.when(mc > 0)
            def _():
                for h in range(H):
                    for j in range(JH):
                        col = h * HD + j * L
                        acc_b[pl.ds(col, L)] = acc_b[pl.ds(col, L)] * scvs[h]

            _accumulate(0, rem_mc, 0)
            return m_new, denv_new

        m0 = tuple(jnp.float32(NEG) for _ in range(H))
        d0 = tuple(jnp.zeros((L,), jnp.float32) for _ in range(H))
        _, denv_fin = lax.fori_loop(0, n_mc, mc_body, (m0, d0))
        _finalize(denv_fin, 0)
        pltpu.sync_copy(acc_b.at[pl.ds(0, D)],
                        out_hbm.at[pl.ds((obase + ni) * D, D)])

    def _batch(ni, e_lo, k, ec):
        node0 = w * NPT + ni
        start_n = jnp.minimum(node0, N_NODES - NB)
        shift = node0 - start_n
        xr_cp = pltpu.async_copy(
            xrf_hbm.at[pl.ds(start_n * D, NB * D)], xr_b, sem_x)
        off0 = _stage_ids(e_lo)
        cps = [_gather_chunk(off0, cc, ec) for cc in range(ECH)]
        for cp in cps:
            cp.wait()
        xr_cp.wait()

        def node_body(j, _):
            nl = _sv(rp_v, ni + j)
            cnt = _sv(rp_v, ni + j + 1) - nl

            @pl.when(cnt > 0)
            def _():
                local_lo = nl - e_lo
                ab = j * D
                _zero_acc(ab)
                _logits(local_lo, cnt, (shift + j) * D)
                m0 = tuple(jnp.float32(NEG) for _ in range(H))
                d0 = tuple(jnp.zeros((L,), jnp.float32) for _ in range(H))
                _, denv, _ = _weights(m0, d0)
                _accumulate(local_lo, cnt, ab)
                _finalize(denv, ab)

            return 0

        lax.fori_loop(0, k, node_body, 0)
        pltpu.sync_copy(acc_b, out_hbm.at[pl.ds((obase + ni) * D, NB * D)])

    def outer_cond(ni):
        return ni < NPT

    def outer_body(ni):
        e_lo = _sv(rp_v, ni)
        cnt0 = _sv(rp_v, ni + 1) - e_lo
        big = cnt0 > EC

        def k_cond(k):
            return (k < NB) & (_sv(rp_v, ni + k + 1) - e_lo <= EC)

        k = lax.while_loop(k_cond, lambda k: k + 1, jnp.int32(1))
        ec = _sv(rp_v, ni + k) - e_lo

        @pl.when(big)
        def _():
            _big_node(ni, e_lo, cnt0)

        @pl.when(jnp.logical_not(big))
        def _():
            _batch(ni, e_lo, k, ec)

        return ni + k

    lax.while_loop(outer_cond, outer_body, jnp.int32(0))


def _gat_edge_sc(xl, xr, srcp, rp_tiles, att_flat, bias, H, EC, NB):
    D = xl.shape[1]
    mesh = plsc.VectorSubcoreMesh(core_axis_name="c", subcore_axis_name="s")
    kfn = pl.kernel(
        functools.partial(_edge_body, D, H, EC, NB),
        out_type=jax.ShapeDtypeStruct((NTILES * PTS * D,), jnp.float32),
        mesh=mesh,
        compiler_params=pltpu.CompilerParams(needs_layout_passes=False),
        scratch_types=[
            pltpu.VMEM((RPW,), jnp.int32),
            pltpu.VMEM((D,), jnp.float32),
            pltpu.VMEM((D,), jnp.float32),
            pltpu.VMEM((D,), jnp.float32),
            pltpu.VMEM((NB * D,), jnp.float32),
            pltpu.VMEM((2 * EC,), jnp.int32),
            pltpu.VMEM((EC, D), jnp.float32),
            pltpu.VMEM((H, 128), jnp.float32),
            pltpu.VMEM((H, 128), jnp.float32),
            pltpu.VMEM((NB * D,), jnp.float32),
            pltpu.SemaphoreType.DMA,
            pltpu.SemaphoreType.DMA,
        ],
    )
    outp = kfn(xl, xr.reshape(-1), srcp, rp_tiles,
               0.6 * att_flat, 0.4 * att_flat, bias)
    outp = outp.reshape(NTILES * PTS, D)
    n = jnp.arange(N_NODES)
    return outp[(n // NPT) * PTS + (n % NPT)]


def kernel(node_features, edge_index, W1l, b1l, W1r, b1r, att1, bias1,
           W2l, b2l, W2r, b2r, att2, bias2):
    N = node_features.shape[0]
    E = edge_index.shape[1]
    ET = E + N

    loop = jnp.arange(N, dtype=jnp.int32)
    src = jnp.concatenate([edge_index[0].astype(jnp.int32), loop])
    dst = jnp.concatenate([edge_index[1].astype(jnp.int32), loop])
    packed = jnp.sort((dst << 17) | jnp.arange(ET, dtype=jnp.int32))
    src_s = src[packed & 0x1FFFF]
    dst_s = packed >> 17
    rowptr = jnp.searchsorted(dst_s, jnp.arange(N + 1)).astype(jnp.int32)

    EP = ET + 272
    srcp = jnp.zeros((EP,), jnp.int32).at[:ET].set(src_s)
    node_idx = jnp.minimum(
        jnp.arange(NTILES)[:, None] * NPT + jnp.arange(RPW)[None, :], N)
    rp_tiles = rowptr[node_idx]

    w1 = jnp.concatenate([W1l, W1r], axis=1)
    bb1 = jnp.concatenate([b1l, b1r])
    xlr = _matmul_bias(node_features, w1, bb1)
    xl1, xr1 = xlr[:, :H1 * HID], xlr[:, H1 * HID:]
    h = _gat_edge_sc(xl1, xr1, srcp, rp_tiles, att1.reshape(-1), bias1,
                     H1, 64, 8)

    w2 = jnp.concatenate([W2l, W2r], axis=1)
    bb2 = jnp.concatenate([b2l, b2r])
    xlr2 = _matmul_bias(h, w2, bb2)
    xl2, xr2 = xlr2[:, :H2 * HID], xlr2[:, H2 * HID:]
    return _gat_edge_sc(xl2, xr2, srcp, rp_tiles, att2.reshape(-1), bias2,
                        H2, 128, 16)


# speedup vs baseline: 1.3592x; 1.3592x over previous
import functools

import jax
import jax.numpy as jnp
from jax import lax
from jax.experimental import pallas as pl
from jax.experimental.pallas import tpu as pltpu
from jax.experimental.pallas import tpu_sc as plsc

N_NODES = 10000
HID = 256
H1 = 4
H2 = 1

L = 16
NTILES = 32
NPT = 313
RPW = 352
PTS = NPT + 16
NEG = -1e30


def _mm_kernel(x_ref, w_ref, b_ref, o_ref):
    o_ref[...] = jnp.dot(x_ref[...], w_ref[...],
                         preferred_element_type=jnp.float32) + b_ref[...]


def _matmul_bias(x, w, b, block_m=400):
    M, K = x.shape
    _, Nf = w.shape
    return pl.pallas_call(
        _mm_kernel,
        grid=(M // block_m,),
        in_specs=[
            pl.BlockSpec((block_m, K), lambda i: (i, 0)),
            pl.BlockSpec((K, Nf), lambda i: (0, 0)),
            pl.BlockSpec((1, Nf), lambda i: (0, 0)),
        ],
        out_specs=pl.BlockSpec((block_m, Nf), lambda i: (i, 0)),
        out_shape=jax.ShapeDtypeStruct((M, Nf), jnp.float32),
    )(x, w, b[None])


def _edge_body(D, H, EC, NB, xl_hbm, xrf_hbm, srcp_hbm, rpt_hbm, att6_hbm,
               att4_hbm, bias_hbm, out_hbm, rp_v, att6_v, att4_v, bias_v,
               xr_b, ids_v, rows_v, lbuf_v, wbuf_v, acc_b, sem_g, sem_x):
    HD = D // H
    JH = HD // L
    JD = D // L
    ECH = EC // L

    c = lax.axis_index("c")
    s = lax.axis_index("s")
    w = s * 2 + c
    obase = w * PTS

    pltpu.sync_copy(rpt_hbm.at[w], rp_v)
    pltpu.sync_copy(att6_hbm, att6_v)
    pltpu.sync_copy(att4_hbm, att4_v)
    pltpu.sync_copy(bias_hbm, bias_v)
    lanes = lax.iota(jnp.int32, L)

    def _sv(ref, i):
        return ref[pl.ds(i, L)][0]

    def _stage_ids(e_lo):
        al = pl.multiple_of((e_lo // 8) * 8, 8)
        pltpu.sync_copy(srcp_hbm.at[pl.ds(al, EC + 8)],
                        ids_v.at[pl.ds(0, EC + 8)])
        return e_lo - al

    def _gather_chunk(off0, c0, valid):
        pos = c0 * L + lanes
        mask = pos < valid
        offs = jnp.where(mask, off0 + pos, 0)
        ids = plsc.load_gather(ids_v, [offs])
        ids = jnp.where(mask, ids, 0)
        return pltpu.async_copy(xl_hbm.at[ids],
                                rows_v.at[pl.ds(c0 * L, L)], sem_g)

    def _logits(local_lo, cnt, xrb):
        for h in range(H):
            for cc in range(ECH):
                lbuf_v[h, cc * L:(cc + 1) * L] = jnp.full(
                    (L,), NEG, jnp.float32)
        n_ch = (cnt + L - 1) // L

        def ch_body(ch, _):
            cbase = ch * L

            def e_body(el, lvecs):
                er = local_lo + cbase + el
                new = []
                for h in range(H):
                    acc = jnp.zeros((L,), jnp.float32)
                    for j in range(JH):
                        col = h * HD + j * L
                        sv = (rows_v[er, col:col + L]
                              + xr_b[pl.ds(xrb + col, L)])
                        acc = (acc + att6_v[col:col + L] * sv
                               + att4_v[col:col + L] * jnp.abs(sv))
                    lg = jnp.sum(acc)
                    new.append(jnp.where(lanes == el, lg, lvecs[h]))
                return tuple(new)

            lvecs = lax.fori_loop(
                0, jnp.minimum(cnt - cbase, L), e_body,
                tuple(jnp.full((L,), NEG, jnp.float32) for _ in range(H)))
            for h in range(H):
                lbuf_v[h, pl.ds(cbase, L)] = lvecs[h]
            return 0

        lax.fori_loop(0, n_ch, ch_body, 0)

    def _weights(m_old, denv_old):
        m_new, denv_new, scvs = [], [], []
        for h in range(H):
            mh = m_old[h]
            for cc in range(ECH):
                mh = jnp.maximum(mh, jnp.max(lbuf_v[h, cc * L:(cc + 1) * L]))
            scv = jnp.exp(jnp.full((L,), m_old[h] - mh))
            denv = denv_old[h] * scv
            for cc in range(ECH):
                wv = jnp.exp(lbuf_v[h, cc * L:(cc + 1) * L] - mh)
                wbuf_v[h, cc * L:(cc + 1) * L] = wv
                denv = denv + wv
            m_new.append(mh)
            denv_new.append(denv)
            scvs.append(scv)
        return tuple(m_new), tuple(denv_new), tuple(scvs)

    def _accumulate(local_lo, cnt, ab):
        def e2_body(el, _):
            er = local_lo + el
            for h in range(H):
                wv = plsc.load_gather(
                    wbuf_v,
                    [jnp.full((L,), h, jnp.int32),
                     jnp.full((L,), el, jnp.int32)])
                for j in range(JH):
                    col = h * HD + j * L
                    acc_b[pl.ds(ab + col, L)] = (
                        acc_b[pl.ds(ab + col, L)]
                        + wv * rows_v[er, col:col + L])
            return 0

        lax.fori_loop(0, cnt, e2_body, 0)

    def _finalize(denv_fin, ab):
        for h in range(H):
            inv = jnp.full((L,), 1.0) / jnp.full((L,), jnp.sum(denv_fin[h]))
            for j in range(JH):
                col = h * HD + j * L
                ov = acc_b[pl.ds(ab + col, L)] * inv + bias_v[col:col + L]
                acc_b[pl.ds(ab + col, L)] = jnp.maximum(ov, 0.0)

    def _zero_acc(ab):
        for j in range(JD):
            acc_b[pl.ds(ab + j * L, L)] = jnp.zeros((L,), jnp.float32)

    def _big_node(ni, e_lo, cnt):
        node = w * NPT + ni
        pltpu.async_copy(xrf_hbm.at[pl.ds(node * D, D)],
                         xr_b.at[pl.ds(0, D)], sem_x).wait()
        _zero_acc(0)
        n_mc = (cnt + EC - 1) // EC

        def mc_body(mc, mc_carry):
            m_old, denv_old = mc_carry
            e_base = e_lo + mc * EC
            rem_mc = jnp.minimum(cnt - mc * EC, EC)
            off0 = _stage_ids(e_base)
            cps = [_gather_chunk(off0, cc, rem_mc) for cc in range(ECH)]
            for cp in cps:
                cp.wait()
            _logits(0, rem_mc, 0)
            m_new, denv_new, scvs = _weights(m_old, denv_old)

            @pl.when(mc > 0)
            def _():
                for h in range(H):
                    for j in range(JH):
                        col = h * HD + j * L
                        acc_b[pl.ds(col, L)] = acc_b[pl.ds(col, L)] * scvs[h]

            _accumulate(0, rem_mc, 0)
            return m_new, denv_new

        m0 = tuple(jnp.float32(NEG) for _ in range(H))
        d0 = tuple(jnp.zeros((L,), jnp.float32) for _ in range(H))
        _, denv_fin = lax.fori_loop(0, n_mc, mc_body, (m0, d0))
        _finalize(denv_fin, 0)
        pltpu.sync_copy(acc_b.at[pl.ds(0, D)],
                        out_hbm.at[pl.ds((obase + ni) * D, D)])

    def _batch(ni, e_lo, k, ec):
        node0 = w * NPT + ni
        start_n = jnp.minimum(node0, N_NODES - NB)
        shift = node0 - start_n
        xr_cp = pltpu.async_copy(
            xrf_hbm.at[pl.ds(start_n * D, NB * D)], xr_b, sem_x)
        off0 = _stage_ids(e_lo)
        cps = [_gather_chunk(off0, cc, ec) for cc in range(ECH)]
        for cp in cps:
            cp.wait()
        xr_cp.wait()

        def node_body(j, _):
            nl = _sv(rp_v, ni + j)
            cnt = _sv(rp_v, ni + j + 1) - nl

            @pl.when(cnt > 0)
            def _():
                local_lo = nl - e_lo
                ab = j * D
                _zero_acc(ab)
                _logits(local_lo, cnt, (shift + j) * D)
                m0 = tuple(jnp.float32(NEG) for _ in range(H))
                d0 = tuple(jnp.zeros((L,), jnp.float32) for _ in range(H))
                _, denv, _ = _weights(m0, d0)
                _accumulate(local_lo, cnt, ab)
                _finalize(denv, ab)

            return 0

        lax.fori_loop(0, k, node_body, 0)
        pltpu.sync_copy(acc_b, out_hbm.at[pl.ds((obase + ni) * D, NB * D)])

    def outer_cond(ni):
        return ni < NPT

    def outer_body(ni):
        e_lo = _sv(rp_v, ni)
        cnt0 = _sv(rp_v, ni + 1) - e_lo
        big = cnt0 > EC

        def k_cond(k):
            return (k < NB) & (_sv(rp_v, ni + k + 1) - e_lo <= EC)

        k = lax.while_loop(k_cond, lambda k: k + 1, jnp.int32(1))
        ec = _sv(rp_v, ni + k) - e_lo

        @pl.when(big)
        def _():
            _big_node(ni, e_lo, cnt0)

        @pl.when(jnp.logical_not(big))
        def _():
            _batch(ni, e_lo, k, ec)

        return ni + k

    lax.while_loop(outer_cond, outer_body, jnp.int32(0))


def _gat_edge_sc(xl, xr, srcp, rp_tiles, att_flat, bias, H, EC, NB):
    D = xl.shape[1]
    mesh = plsc.VectorSubcoreMesh(core_axis_name="c", subcore_axis_name="s")
    kfn = pl.kernel(
        functools.partial(_edge_body, D, H, EC, NB),
        out_type=jax.ShapeDtypeStruct((NTILES * PTS * D,), jnp.float32),
        mesh=mesh,
        compiler_params=pltpu.CompilerParams(needs_layout_passes=False),
        scratch_types=[
            pltpu.VMEM((RPW,), jnp.int32),
            pltpu.VMEM((D,), jnp.float32),
            pltpu.VMEM((D,), jnp.float32),
            pltpu.VMEM((D,), jnp.float32),
            pltpu.VMEM((NB * D,), jnp.float32),
            pltpu.VMEM((2 * EC,), jnp.int32),
            pltpu.VMEM((EC, D), jnp.float32),
            pltpu.VMEM((H, 128), jnp.float32),
            pltpu.VMEM((H, 128), jnp.float32),
            pltpu.VMEM((NB * D,), jnp.float32),
            pltpu.SemaphoreType.DMA,
            pltpu.SemaphoreType.DMA,
        ],
    )
    outp = kfn(xl, xr.reshape(-1), srcp, rp_tiles,
               0.6 * att_flat, 0.4 * att_flat, bias)
    outp = outp.reshape(NTILES * PTS, D)
    n = jnp.arange(N_NODES)
    return outp[(n // NPT) * PTS + (n % NPT)]


def kernel(node_features, edge_index, W1l, b1l, W1r, b1r, att1, bias1,
           W2l, b2l, W2r, b2r, att2, bias2):
    N = node_features.shape[0]
    E = edge_index.shape[1]
    ET = E + N

    loop = jnp.arange(N, dtype=jnp.int32)
    src = jnp.concatenate([edge_index[0].astype(jnp.int32), loop])
    dst = jnp.concatenate([edge_index[1].astype(jnp.int32), loop])
    packed = jnp.sort((dst << 17) | jnp.arange(ET, dtype=jnp.int32))
    src_s = src[packed & 0x1FFFF]
    dst_s = packed >> 17
    rowptr = jnp.searchsorted(dst_s, jnp.arange(N + 1)).astype(jnp.int32)

    EP = ET + 272
    srcp = jnp.zeros((EP,), jnp.int32).at[:ET].set(src_s)
    node_idx = jnp.minimum(
        jnp.arange(NTILES)[:, None] * NPT + jnp.arange(RPW)[None, :], N)
    rp_tiles = rowptr[node_idx]

    w1 = jnp.concatenate([W1l, W1r], axis=1)
    bb1 = jnp.concatenate([b1l, b1r])
    xlr = _matmul_bias(node_features, w1, bb1)
    xl1, xr1 = xlr[:, :H1 * HID], xlr[:, H1 * HID:]
    h = _gat_edge_sc(xl1, xr1, srcp, rp_tiles, att1.reshape(-1), bias1,
                     H1, 64, 8)

    w2 = jnp.concatenate([W2l, W2r], axis=1)
    bb2 = jnp.concatenate([b2l, b2r])
    xlr2 = _matmul_bias(h, w2, bb2)
    xl2, xr2 = xlr2[:, :H2 * HID], xlr2[:, H2 * HID:]
    return _gat_edge_sc(xl2, xr2, srcp, rp_tiles, att2.reshape(-1), bias2,
                        H2, 128, 16)


# speedup vs baseline: 1.4455x; 1.0635x over previous
import functools

import jax
import jax.numpy as jnp
from jax import lax
from jax.experimental import pallas as pl
from jax.experimental.pallas import tpu as pltpu
from jax.experimental.pallas import tpu_sc as plsc

N_NODES = 10000
HID = 256
H1 = 4
H2 = 1

L = 16
NTILES = 32
NPT = 313
RPW = 352
PTS = NPT + 16
NEG = -1e30


def _mm_kernel(x_ref, w_ref, b_ref, o_ref):
    o_ref[...] = jnp.dot(x_ref[...], w_ref[...],
                         preferred_element_type=jnp.float32) + b_ref[...]


def _matmul_bias(x, w, b, block_m=400):
    M, K = x.shape
    _, Nf = w.shape
    return pl.pallas_call(
        _mm_kernel,
        grid=(M // block_m,),
        in_specs=[
            pl.BlockSpec((block_m, K), lambda i: (i, 0)),
            pl.BlockSpec((K, Nf), lambda i: (0, 0)),
            pl.BlockSpec((1, Nf), lambda i: (0, 0)),
        ],
        out_specs=pl.BlockSpec((block_m, Nf), lambda i: (i, 0)),
        out_shape=jax.ShapeDtypeStruct((M, Nf), jnp.float32),
    )(x, w, b[None])


def _edge_body(D, H, EC, NB, xl_hbm, xrf_hbm, srcp_hbm, rpt_hbm, att_hbm,
               bias_hbm, out_hbm, rp_v, att_v, bias_v,
               xr_b, ids_v, rows_v, lbuf_v, wbuf_v, acc_b, sem_g, sem_x):
    HD = D // H
    JH = HD // L
    JD = D // L
    ECH = EC // L

    c = lax.axis_index("c")
    s = lax.axis_index("s")
    w = s * 2 + c
    obase = w * PTS

    pltpu.sync_copy(rpt_hbm.at[w], rp_v)
    pltpu.sync_copy(att_hbm, att_v)
    pltpu.sync_copy(bias_hbm, bias_v)
    lanes = lax.iota(jnp.int32, L)

    def _sv(ref, i):
        return ref[pl.ds(i, L)][0]

    def _stage_ids(e_lo):
        al = pl.multiple_of((e_lo // 8) * 8, 8)
        pltpu.sync_copy(srcp_hbm.at[pl.ds(al, EC + 8)],
                        ids_v.at[pl.ds(0, EC + 8)])
        return e_lo - al

    def _gather_chunk(off0, c0, valid):
        pos = c0 * L + lanes
        mask = pos < valid
        offs = jnp.where(mask, off0 + pos, 0)
        ids = plsc.load_gather(ids_v, [offs])
        ids = jnp.where(mask, ids, 0)
        return pltpu.async_copy(xl_hbm.at[ids],
                                rows_v.at[pl.ds(c0 * L, L)], sem_g)

    def _logits(local_lo, cnt, xrb):
        for h in range(H):
            for cc in range(ECH):
                lbuf_v[h, cc * L:(cc + 1) * L] = jnp.full(
                    (L,), NEG, jnp.float32)
        n_ch = (cnt + L - 1) // L

        def ch_body(ch, _):
            cbase = ch * L

            def e_body(el, lvecs):
                er = local_lo + cbase + el
                new = []
                for h in range(H):
                    acc = jnp.zeros((L,), jnp.float32)
                    for j in range(JH):
                        col = h * HD + j * L
                        sv = (rows_v[er, col:col + L]
                              + xr_b[pl.ds(xrb + col, L)])
                        tv = 0.6 * sv + 0.4 * jnp.abs(sv)
                        acc = acc + att_v[col:col + L] * tv
                    lg = jnp.sum(acc)
                    new.append(jnp.where(lanes == el, lg, lvecs[h]))
                return tuple(new)

            lvecs = lax.fori_loop(
                0, jnp.minimum(cnt - cbase, L), e_body,
                tuple(jnp.full((L,), NEG, jnp.float32) for _ in range(H)))
            for h in range(H):
                lbuf_v[h, pl.ds(cbase, L)] = lvecs[h]
            return 0

        lax.fori_loop(0, n_ch, ch_body, 0)

    def _weights(m_old, denv_old):
        m_new, denv_new, scvs = [], [], []
        for h in range(H):
            mh = m_old[h]
            for cc in range(ECH):
                mh = jnp.maximum(mh, jnp.max(lbuf_v[h, cc * L:(cc + 1) * L]))
            scv = jnp.exp(jnp.full((L,), m_old[h] - mh))
            denv = denv_old[h] * scv
            for cc in range(ECH):
                wv = jnp.exp(lbuf_v[h, cc * L:(cc + 1) * L] - mh)
                wbuf_v[h, cc * L:(cc + 1) * L] = wv
                denv = denv + wv
            m_new.append(mh)
            denv_new.append(denv)
            scvs.append(scv)
        return tuple(m_new), tuple(denv_new), tuple(scvs)

    def _accumulate(local_lo, cnt, ab):
        def e2_body(el, _):
            er = local_lo + el
            for h in range(H):
                wv = plsc.load_gather(
                    wbuf_v,
                    [jnp.full((L,), h, jnp.int32),
                     jnp.full((L,), el, jnp.int32)])
                for j in range(JH):
                    col = h * HD + j * L
                    acc_b[pl.ds(ab + col, L)] = (
                        acc_b[pl.ds(ab + col, L)]
                        + wv * rows_v[er, col:col + L])
            return 0

        lax.fori_loop(0, cnt, e2_body, 0)

    def _finalize(denv_fin, ab):
        for h in range(H):
            inv = jnp.full((L,), 1.0) / jnp.full((L,), jnp.sum(denv_fin[h]))
            for j in range(JH):
                col = h * HD + j * L
                ov = acc_b[pl.ds(ab + col, L)] * inv + bias_v[col:col + L]
                acc_b[pl.ds(ab + col, L)] = jnp.maximum(ov, 0.0)

    def _zero_acc(ab):
        for j in range(JD):
            acc_b[pl.ds(ab + j * L, L)] = jnp.zeros((L,), jnp.float32)

    def _big_node(ni, e_lo, cnt):
        node = w * NPT + ni
        pltpu.async_copy(xrf_hbm.at[pl.ds(node * D, D)],
                         xr_b.at[pl.ds(0, D)], sem_x).wait()
        _zero_acc(0)
        n_mc = (cnt + EC - 1) // EC

        def mc_body(mc, mc_carry):
            m_old, denv_old = mc_carry
            e_base = e_lo + mc * EC
            rem_mc = jnp.minimum(cnt - mc * EC, EC)
            off0 = _stage_ids(e_base)
            cps = [_gather_chunk(off0, cc, rem_mc) for cc in range(ECH)]
            for cp in cps:
                cp.wait()
            _logits(0, rem_mc, 0)
            m_new, denv_new, scvs = _weights(m_old, denv_old)

            @pl.when(mc > 0)
            def _():
                for h in range(H):
                    for j in range(JH):
                        col = h * HD + j * L
                        acc_b[pl.ds(col, L)] = acc_b[pl.ds(col, L)] * scvs[h]

            _accumulate(0, rem_mc, 0)
            return m_new, denv_new

        m0 = tuple(jnp.float32(NEG) for _ in range(H))
        d0 = tuple(jnp.zeros((L,), jnp.float32) for _ in range(H))
        _, denv_fin = lax.fori_loop(0, n_mc, mc_body, (m0, d0))
        _finalize(denv_fin, 0)
        pltpu.sync_copy(acc_b.at[pl.ds(0, D)],
                        out_hbm.at[pl.ds((obase + ni) * D, D)])

    def _batch(ni, e_lo, k, ec):
        node0 = w * NPT + ni
        start_n = jnp.minimum(node0, N_NODES - NB)
        shift = node0 - start_n
        xr_cp = pltpu.async_copy(
            xrf_hbm.at[pl.ds(start_n * D, NB * D)], xr_b, sem_x)
        off0 = _stage_ids(e_lo)
        cps = [_gather_chunk(off0, cc, ec) for cc in range(ECH)]
        for cp in cps:
            cp.wait()
        xr_cp.wait()

        def node_body(j, _):
            nl = _sv(rp_v, ni + j)
            cnt = _sv(rp_v, ni + j + 1) - nl

            @pl.when(cnt > 0)
            def _():
                local_lo = nl - e_lo
                ab = j * D
                _zero_acc(ab)
                _logits(local_lo, cnt, (shift + j) * D)
                m0 = tuple(jnp.float32(NEG) for _ in range(H))
                d0 = tuple(jnp.zeros((L,), jnp.float32) for _ in range(H))
                _, denv, _ = _weights(m0, d0)
                _accumulate(local_lo, cnt, ab)
                _finalize(denv, ab)

            return 0

        lax.fori_loop(0, k, node_body, 0)
        pltpu.sync_copy(acc_b, out_hbm.at[pl.ds((obase + ni) * D, NB * D)])

    def outer_cond(ni):
        return ni < NPT

    def outer_body(ni):
        e_lo = _sv(rp_v, ni)
        cnt0 = _sv(rp_v, ni + 1) - e_lo
        big = cnt0 > EC

        def k_cond(k):
            return (k < NB) & (_sv(rp_v, ni + k + 1) - e_lo <= EC)

        k = lax.while_loop(k_cond, lambda k: k + 1, jnp.int32(1))
        ec = _sv(rp_v, ni + k) - e_lo

        @pl.when(big)
        def _():
            _big_node(ni, e_lo, cnt0)

        @pl.when(jnp.logical_not(big))
        def _():
            _batch(ni, e_lo, k, ec)

        return ni + k

    lax.while_loop(outer_cond, outer_body, jnp.int32(0))


def _gat_edge_sc(xl, xr, srcp, rp_tiles, att_flat, bias, H, EC, NB):
    D = xl.shape[1]
    mesh = plsc.VectorSubcoreMesh(core_axis_name="c", subcore_axis_name="s")
    kfn = pl.kernel(
        functools.partial(_edge_body, D, H, EC, NB),
        out_type=jax.ShapeDtypeStruct((NTILES * PTS * D,), jnp.float32),
        mesh=mesh,
        compiler_params=pltpu.CompilerParams(needs_layout_passes=False),
        scratch_types=[
            pltpu.VMEM((RPW,), jnp.int32),
            pltpu.VMEM((D,), jnp.float32),
            pltpu.VMEM((D,), jnp.float32),
            pltpu.VMEM((NB * D,), jnp.float32),
            pltpu.VMEM((2 * EC,), jnp.int32),
            pltpu.VMEM((EC, D), jnp.float32),
            pltpu.VMEM((H, 128), jnp.float32),
            pltpu.VMEM((H, 128), jnp.float32),
            pltpu.VMEM((NB * D,), jnp.float32),
            pltpu.SemaphoreType.DMA,
            pltpu.SemaphoreType.DMA,
        ],
    )
    outp = kfn(xl, xr.reshape(-1), srcp, rp_tiles, att_flat, bias)
    outp = outp.reshape(NTILES * PTS, D)
    n = jnp.arange(N_NODES)
    return outp[(n // NPT) * PTS + (n % NPT)]


def kernel(node_features, edge_index, W1l, b1l, W1r, b1r, att1, bias1,
           W2l, b2l, W2r, b2r, att2, bias2):
    N = node_features.shape[0]
    E = edge_index.shape[1]
    ET = E + N

    loop = jnp.arange(N, dtype=jnp.int32)
    src = jnp.concatenate([edge_index[0].astype(jnp.int32), loop])
    dst = jnp.concatenate([edge_index[1].astype(jnp.int32), loop])
    packed = jnp.sort((dst << 17) | jnp.arange(ET, dtype=jnp.int32))
    src_s = src[packed & 0x1FFFF]
    dst_s = packed >> 17
    rowptr = jnp.searchsorted(dst_s, jnp.arange(N + 1)).astype(jnp.int32)

    EP = ET + 272
    srcp = jnp.zeros((EP,), jnp.int32).at[:ET].set(src_s)
    node_idx = jnp.minimum(
        jnp.arange(NTILES)[:, None] * NPT + jnp.arange(RPW)[None, :], N)
    rp_tiles = rowptr[node_idx]

    w1 = jnp.concatenate([W1l, W1r], axis=1)
    bb1 = jnp.concatenate([b1l, b1r])
    xlr = _matmul_bias(node_features, w1, bb1)
    xl1, xr1 = xlr[:, :H1 * HID], xlr[:, H1 * HID:]
    h = _gat_edge_sc(xl1, xr1, srcp, rp_tiles, att1.reshape(-1), bias1,
                     H1, 64, 8)

    w2 = jnp.concatenate([W2l, W2r], axis=1)
    bb2 = jnp.concatenate([b2l, b2r])
    xlr2 = _matmul_bias(h, w2, bb2)
    xl2, xr2 = xlr2[:, :H2 * HID], xlr2[:, H2 * HID:]
    return _gat_edge_sc(xl2, xr2, srcp, rp_tiles, att2.reshape(-1), bias2,
                        H2, 128, 16)


# speedup vs baseline: 1.4795x; 1.0236x over previous
import functools

import jax
import jax.numpy as jnp
from jax import lax
from jax.experimental import pallas as pl
from jax.experimental.pallas import tpu as pltpu
from jax.experimental.pallas import tpu_sc as plsc

N_NODES = 10000
HID = 256
H1 = 4
H2 = 1

L = 16
NTILES = 32
NPT = 313
RPW = 352
PTS = NPT + 16
NEG = -1e30


def _mm_kernel(x_ref, w_ref, b_ref, o_ref):
    o_ref[...] = jnp.dot(x_ref[...], w_ref[...],
                         preferred_element_type=jnp.float32) + b_ref[...]


def _matmul_bias(x, w, b, block_m=400):
    M, K = x.shape
    _, Nf = w.shape
    return pl.pallas_call(
        _mm_kernel,
        grid=(M // block_m,),
        in_specs=[
            pl.BlockSpec((block_m, K), lambda i: (i, 0)),
            pl.BlockSpec((K, Nf), lambda i: (0, 0)),
            pl.BlockSpec((1, Nf), lambda i: (0, 0)),
        ],
        out_specs=pl.BlockSpec((block_m, Nf), lambda i: (i, 0)),
        out_shape=jax.ShapeDtypeStruct((M, Nf), jnp.float32),
    )(x, w, b[None])


def _edge_body(D, H, EC, NB, xl_hbm, xrf_hbm, srcp_hbm, rpt_hbm, att_hbm,
               bias_hbm, out_hbm, rp_v, att_v, bias_v,
               xr_b, ids_v, rows_v, lbuf_v, wbuf_v, acc_b, sem_g, sem_x):
    HD = D // H
    JH = HD // L
    JD = D // L
    ECH = EC // L

    c = lax.axis_index("c")
    s = lax.axis_index("s")
    w = s * 2 + c
    obase = w * PTS

    pltpu.sync_copy(rpt_hbm.at[w], rp_v)
    pltpu.sync_copy(att_hbm, att_v)
    pltpu.sync_copy(bias_hbm, bias_v)
    lanes = lax.iota(jnp.int32, L)

    def _sv(ref, i):
        return ref[pl.ds(i, L)][0]

    def _stage_ids(e_lo):
        al = pl.multiple_of((e_lo // 8) * 8, 8)
        pltpu.sync_copy(srcp_hbm.at[pl.ds(al, EC + 8)],
                        ids_v.at[pl.ds(0, EC + 8)])
        return e_lo - al

    def _gather_chunk(off0, c0, valid):
        pos = c0 * L + lanes
        mask = pos < valid
        offs = jnp.where(mask, off0 + pos, 0)
        ids = plsc.load_gather(ids_v, [offs])
        ids = jnp.where(mask, ids, 0)
        return pltpu.async_copy(xl_hbm.at[ids],
                                rows_v.at[pl.ds(c0 * L, L)], sem_g)

    def _logits(local_lo, cnt, xrb):
        for h in range(H):
            for cc in range(ECH):
                lbuf_v[h, cc * L:(cc + 1) * L] = jnp.full(
                    (L,), NEG, jnp.float32)
        n_ch = (cnt + L - 1) // L

        def ch_body(ch, _):
            cbase = ch * L

            def e_body(el, lvecs):
                er = local_lo + cbase + el
                new = []
                for h in range(H):
                    acc = jnp.zeros((L,), jnp.float32)
                    for j in range(JH):
                        col = h * HD + j * L
                        sv = (rows_v[er, col:col + L]
                              + xr_b[pl.ds(xrb + col, L)])
                        tv = 0.6 * sv + 0.4 * jnp.abs(sv)
                        acc = acc + att_v[col:col + L] * tv
                    lg = jnp.sum(acc)
                    new.append(jnp.where(lanes == el, lg, lvecs[h]))
                return tuple(new)

            lvecs = lax.fori_loop(
                0, jnp.minimum(cnt - cbase, L), e_body,
                tuple(jnp.full((L,), NEG, jnp.float32) for _ in range(H)))
            for h in range(H):
                lbuf_v[h, pl.ds(cbase, L)] = lvecs[h]
            return 0

        lax.fori_loop(0, n_ch, ch_body, 0)

    def _weights(m_old, denv_old):
        m_new, denv_new, scvs = [], [], []
        for h in range(H):
            mh = m_old[h]
            for cc in range(ECH):
                mh = jnp.maximum(mh, jnp.max(lbuf_v[h, cc * L:(cc + 1) * L]))
            scv = jnp.exp(jnp.full((L,), m_old[h] - mh))
            denv = denv_old[h] * scv
            for cc in range(ECH):
                wv = jnp.exp(lbuf_v[h, cc * L:(cc + 1) * L] - mh)
                wbuf_v[h, cc * L:(cc + 1) * L] = wv
                denv = denv + wv
            m_new.append(mh)
            denv_new.append(denv)
            scvs.append(scv)
        return tuple(m_new), tuple(denv_new), tuple(scvs)

    def _accumulate(local_lo, cnt, ab):
        def e2_body(el, _):
            er = local_lo + el
            for h in range(H):
                wv = plsc.load_gather(
                    wbuf_v,
                    [jnp.full((L,), h, jnp.int32),
                     jnp.full((L,), el, jnp.int32)])
                for j in range(JH):
                    col = h * HD + j * L
                    acc_b[pl.ds(ab + col, L)] = (
                        acc_b[pl.ds(ab + col, L)]
                        + wv * rows_v[er, col:col + L])
            return 0

        lax.fori_loop(0, cnt, e2_body, 0)

    def _finalize(denv_fin, ab):
        for h in range(H):
            inv = jnp.full((L,), 1.0) / jnp.full((L,), jnp.sum(denv_fin[h]))
            for j in range(JH):
                col = h * HD + j * L
                ov = acc_b[pl.ds(ab + col, L)] * inv + bias_v[col:col + L]
                acc_b[pl.ds(ab + col, L)] = jnp.maximum(ov, 0.0)

    def _zero_acc(ab):
        for j in range(JD):
            acc_b[pl.ds(ab + j * L, L)] = jnp.zeros((L,), jnp.float32)

    def _big_node(ni, e_lo, cnt):
        node = w * NPT + ni
        pltpu.async_copy(xrf_hbm.at[pl.ds(node * D, D)],
                         xr_b.at[pl.ds(0, D)], sem_x).wait()
        _zero_acc(0)
        n_mc = (cnt + EC - 1) // EC

        def mc_body(mc, mc_carry):
            m_old, denv_old = mc_carry
            e_base = e_lo + mc * EC
            rem_mc = jnp.minimum(cnt - mc * EC, EC)
            off0 = _stage_ids(e_base)
            cps = [_gather_chunk(off0, cc, rem_mc) for cc in range(ECH)]
            for cp in cps:
                cp.wait()
            _logits(0, rem_mc, 0)
            m_new, denv_new, scvs = _weights(m_old, denv_old)

            @pl.when(mc > 0)
            def _():
                for h in range(H):
                    for j in range(JH):
                        col = h * HD + j * L
                        acc_b[pl.ds(col, L)] = acc_b[pl.ds(col, L)] * scvs[h]

            _accumulate(0, rem_mc, 0)
            return m_new, denv_new

        m0 = tuple(jnp.float32(NEG) for _ in range(H))
        d0 = tuple(jnp.zeros((L,), jnp.float32) for _ in range(H))
        _, denv_fin = lax.fori_loop(0, n_mc, mc_body, (m0, d0))
        _finalize(denv_fin, 0)
        pltpu.sync_copy(acc_b.at[pl.ds(0, D)],
                        out_hbm.at[pl.ds((obase + ni) * D, D)])

    def _batch(ni, e_lo, k, ec):
        node0 = w * NPT + ni
        start_n = jnp.minimum(node0, N_NODES - NB)
        shift = node0 - start_n
        xr_cp = pltpu.async_copy(
            xrf_hbm.at[pl.ds(start_n * D, NB * D)], xr_b, sem_x)
        off0 = _stage_ids(e_lo)
        cps = [_gather_chunk(off0, cc, ec) for cc in range(ECH)]
        for cp in cps:
            cp.wait()
        xr_cp.wait()

        def node_body(j, _):
            nl = _sv(rp_v, ni + j)
            cnt = _sv(rp_v, ni + j + 1) - nl

            @pl.when(cnt > 0)
            def _():
                local_lo = nl - e_lo
                ab = j * D
                _zero_acc(ab)
                _logits(local_lo, cnt, (shift + j) * D)
                m0 = tuple(jnp.float32(NEG) for _ in range(H))
                d0 = tuple(jnp.zeros((L,), jnp.float32) for _ in range(H))
                _, denv, _ = _weights(m0, d0)
                _accumulate(local_lo, cnt, ab)
                _finalize(denv, ab)

            return 0

        lax.fori_loop(0, k, node_body, 0)
        pltpu.sync_copy(acc_b, out_hbm.at[pl.ds((obase + ni) * D, NB * D)])

    def outer_cond(ni):
        return ni < NPT

    def outer_body(ni):
        e_lo = _sv(rp_v, ni)
        cnt0 = _sv(rp_v, ni + 1) - e_lo
        big = cnt0 > EC

        def k_cond(k):
            return (k < NB) & (_sv(rp_v, ni + k + 1) - e_lo <= EC)

        k = lax.while_loop(k_cond, lambda k: k + 1, jnp.int32(1))
        ec = _sv(rp_v, ni + k) - e_lo

        @pl.when(big)
        def _():
            _big_node(ni, e_lo, cnt0)

        @pl.when(jnp.logical_not(big))
        def _():
            _batch(ni, e_lo, k, ec)

        return ni + k

    lax.while_loop(outer_cond, outer_body, jnp.int32(0))


def _gat_edge_sc(xl, xr, srcp, rp_tiles, att_flat, bias, H, EC, NB):
    D = xl.shape[1]
    mesh = plsc.VectorSubcoreMesh(core_axis_name="c", subcore_axis_name="s")
    kfn = pl.kernel(
        functools.partial(_edge_body, D, H, EC, NB),
        out_type=jax.ShapeDtypeStruct((NTILES * PTS * D,), jnp.float32),
        mesh=mesh,
        compiler_params=pltpu.CompilerParams(needs_layout_passes=False),
        scratch_types=[
            pltpu.VMEM((RPW,), jnp.int32),
            pltpu.VMEM((D,), jnp.float32),
            pltpu.VMEM((D,), jnp.float32),
            pltpu.VMEM((NB * D,), jnp.float32),
            pltpu.VMEM((2 * EC,), jnp.int32),
            pltpu.VMEM((EC, D), jnp.float32),
            pltpu.VMEM((H, 128), jnp.float32),
            pltpu.VMEM((H, 128), jnp.float32),
            pltpu.VMEM((NB * D,), jnp.float32),
            pltpu.SemaphoreType.DMA,
            pltpu.SemaphoreType.DMA,
        ],
    )
    outp = kfn(xl, xr.reshape(-1), srcp, rp_tiles, att_flat, bias)
    outp = outp.reshape(NTILES * PTS, D)
    n = jnp.arange(N_NODES)
    return outp[(n // NPT) * PTS + (n % NPT)]


def kernel(node_features, edge_index, W1l, b1l, W1r, b1r, att1, bias1,
           W2l, b2l, W2r, b2r, att2, bias2):
    N = node_features.shape[0]
    E = edge_index.shape[1]
    ET = E + N

    loop = jnp.arange(N, dtype=jnp.int32)
    src = jnp.concatenate([edge_index[0].astype(jnp.int32), loop])
    dst = jnp.concatenate([edge_index[1].astype(jnp.int32), loop])
    order = jnp.argsort(dst)
    src_s = src[order]
    dst_s = dst[order]
    rowptr = jnp.searchsorted(dst_s, jnp.arange(N + 1)).astype(jnp.int32)

    EP = ET + 272
    srcp = jnp.zeros((EP,), jnp.int32).at[:ET].set(src_s)
    node_idx = jnp.minimum(
        jnp.arange(NTILES)[:, None] * NPT + jnp.arange(RPW)[None, :], N)
    rp_tiles = rowptr[node_idx]

    w1 = jnp.concatenate([W1l, W1r], axis=1)
    bb1 = jnp.concatenate([b1l, b1r])
    xlr = _matmul_bias(node_features, w1, bb1)
    xl1, xr1 = xlr[:, :H1 * HID], xlr[:, H1 * HID:]
    h = _gat_edge_sc(xl1, xr1, srcp, rp_tiles, att1.reshape(-1), bias1,
                     H1, 64, 8)

    w2 = jnp.concatenate([W2l, W2r], axis=1)
    bb2 = jnp.concatenate([b2l, b2r])
    xlr2 = _matmul_bias(h, w2, bb2)
    xl2, xr2 = xlr2[:, :H2 * HID], xlr2[:, H2 * HID:]
    return _gat_edge_sc(xl2, xr2, srcp, rp_tiles, att2.reshape(-1), bias2,
                        H2, 128, 16)
